# baseline (speedup 1.0000x reference)
import functools

import jax
import jax.numpy as jnp
from jax import lax
from jax.experimental import pallas as pl
from jax.experimental.pallas import tpu as pltpu
from jax.experimental.pallas import tpu_sc as plsc

NC = 2
NS = 16
NW = NC * NS

B = 8
T = 4
BT = B * T
N = 1601
D = 1280
CK = 16
NFULL = N // CK
VPT = D // 16
UNR = 8
TB = 128


def _tc_local_add(x, local_pe, gate):
  def body(gate_ref, x_ref, l_ref, o_ref):
    c1 = 1.0 - jnp.tanh(gate_ref[0])
    o_ref[...] = x_ref[...] + c1 * l_ref[...][None, None]

  grid = (pl.cdiv(N, TB), B, T)
  return pl.pallas_call(
      body,
      grid=grid,
      in_specs=[
          pl.BlockSpec(memory_space=pltpu.SMEM),
          pl.BlockSpec((1, 1, TB, D), lambda i, j, k: (j, k, i, 0)),
          pl.BlockSpec((TB, D), lambda i, j, k: (i, 0)),
      ],
      out_specs=pl.BlockSpec((1, 1, TB, D), lambda i, j, k: (j, k, i, 0)),
      out_shape=jax.ShapeDtypeStruct((B, T, N, D), jnp.float32),
  )(gate, x, local_pe)


def _tc_relayout_g(global_pe):
  def body(g_ref, o_ref):
    o_ref[...] = g_ref[0]

  grid = (T * T, pl.cdiv(N, TB))
  return pl.pallas_call(
      body,
      grid=grid,
      in_specs=[pl.BlockSpec((1, 1, TB, D), lambda q, i: (q // T, q % T, i, 0))],
      out_specs=pl.BlockSpec((1, TB, D), lambda q, i: (q, i, 0)),
      out_shape=jax.ShapeDtypeStruct((T * T, N, D), jnp.float32),
  )(global_pe)


def _sc_add_global(x3, g3, p_arr, m_arr):
  mesh = plsc.VectorSubcoreMesh(core_axis_name="c", subcore_axis_name="s")

  @functools.partial(
      pl.kernel,
      mesh=mesh,
      out_type=jax.ShapeDtypeStruct((BT, N, D), jnp.float32),
      scratch_types=[
          pltpu.VMEM((CK, D), jnp.float32),
          pltpu.VMEM((CK, D), jnp.float32),
          pltpu.VMEM((CK, D), jnp.float32),
          pltpu.VMEM((CK, D), jnp.float32),
          pltpu.VMEM((BT, 128), jnp.int32),
          pltpu.VMEM((BT, 128), jnp.float32),
          pltpu.SemaphoreType.DMA,
          pltpu.SemaphoreType.DMA,
          pltpu.SemaphoreType.DMA,
          pltpu.SemaphoreType.DMA,
          pltpu.SemaphoreType.DMA,
          pltpu.SemaphoreType.DMA,
      ],
  )
  def k(x_hbm, g_hbm, p_hbm, m_hbm, out_hbm,
        xb0, xb1, gb0, gb1, pb, mb,
        sx0, sx1, sg0, sg1, so0, so1):
    wid = lax.axis_index("s") * NC + lax.axis_index("c")
    pltpu.sync_copy(p_hbm, pb)
    pltpu.sync_copy(m_hbm, mb)
    lo = (wid * NFULL) // NW
    hi = ((wid + 1) * NFULL) // NW

    def drain(buf, sem):
      pltpu.make_async_copy(x_hbm.at[0, pl.ds(0, CK), :], buf, sem).wait()

    def drain_out(buf, sem):
      pltpu.make_async_copy(buf, out_hbm.at[0, pl.ds(0, CK), :], sem).wait()

    def issue(bt, n0, xbuf, gbuf, sx, sg):
      p = pb[bt, pl.ds(0, 16)][0]
      pltpu.async_copy(g_hbm.at[p, pl.ds(n0, CK), :], gbuf, sg)
      pltpu.async_copy(x_hbm.at[bt, pl.ds(n0, CK), :], xbuf, sx)

    def fused_rows(xbuf, gbuf, mv):
      def i_body(i, cc):
        def jj_body(jj, cc2):
          for u in range(UNR):
            s = pl.ds(jj * (16 * UNR) + u * 16, 16)
            xbuf[i, s] = xbuf[i, s] + mv * gbuf[i, s]
          return cc2
        return lax.fori_loop(0, VPT // UNR, jj_body, cc)
      lax.fori_loop(0, CK, i_body, 0)

    def chunk_body(c, carry):
      n0 = c * CK
      issue(0, n0, xb0, gb0, sx0, sg0)

      def pair_body(it2, carry2):
        bt0 = 2 * it2
        bt1 = bt0 + 1

        @pl.when(it2 >= 1)
        def _():
          drain_out(xb1, so1)
        issue(bt1, n0, xb1, gb1, sx1, sg1)

        drain(xb0, sx0)
        drain(gb0, sg0)
        fused_rows(xb0, gb0, mb[bt0, pl.ds(0, 16)])
        pltpu.async_copy(xb0, out_hbm.at[bt0, pl.ds(n0, CK), :], so0)

        @pl.when(it2 <= (BT // 2 - 2))
        def _():
          drain_out(xb0, so0)
          issue(bt0 + 2, n0, xb0, gb0, sx0, sg0)

        drain(xb1, sx1)
        drain(gb1, sg1)
        fused_rows(xb1, gb1, mb[bt1, pl.ds(0, 16)])
        pltpu.async_copy(xb1, out_hbm.at[bt1, pl.ds(n0, CK), :], so1)
        return carry2

      lax.fori_loop(0, BT // 2, pair_body, 0)
      drain_out(xb0, so0)
      drain_out(xb1, so1)
      return carry

    lax.fori_loop(lo, hi, chunk_body, 0)

    @pl.when(wid == 0)
    def _():
      n0 = NFULL * CK

      def bt_tail(bt, carry2):
        p = pb[bt, pl.ds(0, 16)][0]
        cp = pltpu.async_copy(g_hbm.at[p, pl.ds(n0, 1), :],
                              gb0.at[pl.ds(0, 1), :], sg0)
        pltpu.sync_copy(x_hbm.at[bt, pl.ds(n0, 1), :], xb0.at[pl.ds(0, 1), :])
        cp.wait()
        mv = mb[bt, pl.ds(0, 16)]

        def j_body(j, c4):
          s = pl.ds(j * 16, 16)
          xb0[0, s] = xb0[0, s] + mv * gb0[0, s]
          return c4

        lax.fori_loop(0, VPT, j_body, 0)
        pltpu.sync_copy(xb0.at[pl.ds(0, 1), :],
                        out_hbm.at[bt, pl.ds(n0, 1), :])
        return carry2

      lax.fori_loop(0, BT, bt_tail, 0)

  return k(x3, g3, p_arr, m_arr)


def kernel(x, aspect_ratio, local_pe, global_pe, gate):
  b, t, n, d = x.shape
  g2 = jnp.tanh(gate[0].astype(jnp.float32))
  ar = aspect_ratio.astype(jnp.int32)
  h = ar[:, 0]
  w = ar[:, 1]
  wsafe = jnp.maximum(w, 1)
  tt = jnp.arange(T, dtype=jnp.int32)
  rows = tt[None, :] // wsafe[:, None]
  cols = tt[None, :] % wsafe[:, None]
  plane = (rows * T + cols).reshape(BT)
  valid = (tt[None, :] < (h * w)[:, None]).reshape(BT)
  p_arr = jnp.tile(plane.reshape(BT, 1), (1, 128))
  m_arr = jnp.tile((g2 * valid.astype(jnp.float32)).reshape(BT, 1), (1, 128))
  xplus = _tc_local_add(x, local_pe, gate.astype(jnp.float32))
  x3 = xplus.reshape(BT, N, D)
  g3 = _tc_relayout_g(global_pe)
  out = _sc_add_global(x3, g3, p_arr, m_arr)
  return out.reshape(b, t, n, d)


# speedup vs baseline: 1.1427x; 1.1427x over previous
import functools

import jax
import jax.numpy as jnp
from jax import lax
from jax.experimental import pallas as pl
from jax.experimental.pallas import tpu as pltpu
from jax.experimental.pallas import tpu_sc as plsc

NC = 2
NS = 16
NW = NC * NS

B = 8
T = 4
BT = B * T
N = 1601
D = 1280
CK = 16
NFULL = N // CK
VPT = D // 16
UNR = 8
TB = 128


def _tc_local_add(x, local_pe, gate):
  def body(gate_ref, x_ref, l_ref, o_ref):
    c1 = 1.0 - jnp.tanh(gate_ref[0])
    o_ref[...] = x_ref[...] + c1 * l_ref[...][None, None]

  grid = (pl.cdiv(N, TB), B, T)
  return pl.pallas_call(
      body,
      grid=grid,
      in_specs=[
          pl.BlockSpec(memory_space=pltpu.SMEM),
          pl.BlockSpec((1, 1, TB, D), lambda i, j, k: (j, k, i, 0)),
          pl.BlockSpec((TB, D), lambda i, j, k: (i, 0)),
      ],
      out_specs=pl.BlockSpec((1, 1, TB, D), lambda i, j, k: (j, k, i, 0)),
      out_shape=jax.ShapeDtypeStruct((B, T, N, D), jnp.float32),
  )(gate, x, local_pe)


def _sc_add_global(x3, g3, p_arr, m_arr):
  mesh = plsc.VectorSubcoreMesh(core_axis_name="c", subcore_axis_name="s")

  @functools.partial(
      pl.kernel,
      mesh=mesh,
      out_type=jax.ShapeDtypeStruct((BT, N, D), jnp.float32),
      scratch_types=[
          pltpu.VMEM((CK, D), jnp.float32),
          pltpu.VMEM((CK, D), jnp.float32),
          pltpu.VMEM((CK, D), jnp.float32),
          pltpu.VMEM((CK, D), jnp.float32),
          pltpu.VMEM((BT, 128), jnp.int32),
          pltpu.VMEM((BT, 128), jnp.float32),
          pltpu.SemaphoreType.DMA,
          pltpu.SemaphoreType.DMA,
          pltpu.SemaphoreType.DMA,
          pltpu.SemaphoreType.DMA,
          pltpu.SemaphoreType.DMA,
          pltpu.SemaphoreType.DMA,
      ],
  )
  def k(x_hbm, g_hbm, p_hbm, m_hbm, out_hbm,
        xb0, xb1, gb0, gb1, pb, mb,
        sx0, sx1, sg0, sg1, so0, so1):
    wid = lax.axis_index("s") * NC + lax.axis_index("c")
    pltpu.sync_copy(p_hbm, pb)
    pltpu.sync_copy(m_hbm, mb)
    NPW = NFULL * BT // NW
    q0 = wid * NPW

    def drain(buf, sem):
      pltpu.make_async_copy(x_hbm.at[0, pl.ds(0, CK), :], buf, sem).wait()

    def drain_out(buf, sem):
      pltpu.make_async_copy(buf, out_hbm.at[0, pl.ds(0, CK), :], sem).wait()

    def issue(q, xbuf, gbuf, sx, sg):
      bt = lax.rem(q, BT)
      n0 = (q // BT) * CK
      p = pb[bt, pl.ds(0, 16)][0]
      pltpu.async_copy(g_hbm.at[p, pl.ds(n0, CK), :], gbuf, sg)
      pltpu.async_copy(x_hbm.at[bt, pl.ds(n0, CK), :], xbuf, sx)

    def store(q, xbuf, so):
      bt = lax.rem(q, BT)
      n0 = (q // BT) * CK
      pltpu.async_copy(xbuf, out_hbm.at[bt, pl.ds(n0, CK), :], so)

    def fused_rows(xbuf, gbuf, mv):
      def i_body(i, cc):
        def jj_body(jj, cc2):
          for u in range(UNR):
            s = pl.ds(jj * (16 * UNR) + u * 16, 16)
            xbuf[i, s] = xbuf[i, s] + mv * gbuf[i, s]
          return cc2
        return lax.fori_loop(0, VPT // UNR, jj_body, cc)
      lax.fori_loop(0, CK, i_body, 0)

    issue(q0, xb0, gb0, sx0, sg0)

    def pair_body(it2, carry2):
      qa = q0 + 2 * it2
      qb = qa + 1

      @pl.when(it2 >= 1)
      def _():
        drain_out(xb1, so1)
      issue(qb, xb1, gb1, sx1, sg1)

      drain(xb0, sx0)
      drain(gb0, sg0)
      fused_rows(xb0, gb0, mb[lax.rem(qa, BT), pl.ds(0, 16)])
      store(qa, xb0, so0)

      @pl.when(it2 <= (NPW // 2 - 2))
      def _():
        drain_out(xb0, so0)
        issue(qa + 2, xb0, gb0, sx0, sg0)

      drain(xb1, sx1)
      drain(gb1, sg1)
      fused_rows(xb1, gb1, mb[lax.rem(qb, BT), pl.ds(0, 16)])
      store(qb, xb1, so1)
      return carry2

    lax.fori_loop(0, NPW // 2, pair_body, 0)
    drain_out(xb0, so0)
    drain_out(xb1, so1)

    @pl.when(wid == 0)
    def _():
      n0 = NFULL * CK

      def bt_tail(bt, carry2):
        p = pb[bt, pl.ds(0, 16)][0]
        cp = pltpu.async_copy(g_hbm.at[p, pl.ds(n0, 1), :],
                              gb0.at[pl.ds(0, 1), :], sg0)
        pltpu.sync_copy(x_hbm.at[bt, pl.ds(n0, 1), :], xb0.at[pl.ds(0, 1), :])
        cp.wait()
        mv = mb[bt, pl.ds(0, 16)]

        def j_body(j, c4):
          s = pl.ds(j * 16, 16)
          xb0[0, s] = xb0[0, s] + mv * gb0[0, s]
          return c4

        lax.fori_loop(0, VPT, j_body, 0)
        pltpu.sync_copy(xb0.at[pl.ds(0, 1), :],
                        out_hbm.at[bt, pl.ds(n0, 1), :])
        return carry2

      lax.fori_loop(0, BT, bt_tail, 0)

  return k(x3, g3, p_arr, m_arr)


def kernel(x, aspect_ratio, local_pe, global_pe, gate):
  b, t, n, d = x.shape
  g2 = jnp.tanh(gate[0].astype(jnp.float32))
  ar = aspect_ratio.astype(jnp.int32)
  h = ar[:, 0]
  w = ar[:, 1]
  wsafe = jnp.maximum(w, 1)
  tt = jnp.arange(T, dtype=jnp.int32)
  rows = tt[None, :] // wsafe[:, None]
  cols = tt[None, :] % wsafe[:, None]
  plane = (rows * T + cols).reshape(BT)
  valid = (tt[None, :] < (h * w)[:, None]).reshape(BT)
  p_arr = jnp.tile(plane.reshape(BT, 1), (1, 128))
  m_arr = jnp.tile((g2 * valid.astype(jnp.float32)).reshape(BT, 1), (1, 128))
  xplus = _tc_local_add(x, local_pe, gate.astype(jnp.float32))
  x3 = xplus.reshape(BT, N, D)
  g3 = global_pe.reshape(T * T, N, D)
  out = _sc_add_global(x3, g3, p_arr, m_arr)
  return out.reshape(b, t, n, d)
